# Initial kernel scaffold; baseline (speedup 1.0000x reference)
import functools

import jax
import jax.numpy as jnp
from jax import lax
from jax.experimental import pallas as pl
from jax.experimental.pallas import tpu as pltpu
from jax.experimental.pallas import tpu_sc as plsc

N_NODES = 10000
N_EDGES = 320000
D = 128

NC = 2
NS = 16
NW = NC * NS
E_PER_W = N_EDGES // NW
CHUNK = 80
NCHUNK = E_PER_W // CHUNK
ROWS_PER_SUB = N_NODES // NS


def _sc_aggregate(x, src, dst, zeros_rows):
    mesh = plsc.VectorSubcoreMesh(core_axis_name="c", subcore_axis_name="s")

    @functools.partial(
        pl.kernel,
        out_type=jax.ShapeDtypeStruct((NC, N_NODES, D), jnp.float32),
        mesh=mesh,
        scratch_types=[
            pltpu.VMEM((NCHUNK, CHUNK), jnp.int32),
            pltpu.VMEM((NCHUNK, CHUNK), jnp.int32),
            pltpu.VMEM((CHUNK, D), jnp.float32),
            pltpu.VMEM_SHARED((N_NODES, D), jnp.float32),
            pltpu.SemaphoreType.DMA,
        ],
    )
    def agg_kernel(x_hbm, src_hbm, dst_hbm, zeros_hbm, out_hbm,
                   src_v, dst_v, rows_v, acc_sh, sem):
        c = lax.axis_index("c")
        s = lax.axis_index("s")
        wid = c * NS + s

        pltpu.sync_copy(zeros_hbm, acc_sh.at[pl.ds(s * ROWS_PER_SUB, ROWS_PER_SUB)])
        pltpu.sync_copy(src_hbm.at[wid], src_v)
        pltpu.sync_copy(dst_hbm.at[wid], dst_v)
        plsc.subcore_barrier()

        def body(j, carry):
            pltpu.async_copy(x_hbm.at[src_v.at[j]], rows_v, sem).wait()
            pltpu.sync_copy(rows_v, acc_sh.at[dst_v.at[j]], add=True)
            return carry

        lax.fori_loop(0, NCHUNK, body, 0)
        plsc.subcore_barrier()

        pltpu.sync_copy(
            acc_sh.at[pl.ds(s * ROWS_PER_SUB, ROWS_PER_SUB)],
            out_hbm.at[c, pl.ds(s * ROWS_PER_SUB, ROWS_PER_SUB)],
        )

    return agg_kernel(x, src, dst, zeros_rows)


ROW_BLK = 2000
GRID = N_NODES // ROW_BLK


def _mlp_body(eps_ref, x_ref, a0_ref, a1_ref, wa_ref, ba_ref, wb_ref, bb_ref,
              o_ref):
    h = x_ref[...] * (1.0 + eps_ref[0, 0]) + a0_ref[...] + a1_ref[...]
    t = jnp.maximum(
        jnp.dot(h, wa_ref[...], preferred_element_type=jnp.float32)
        + ba_ref[...], 0.0)
    o_ref[...] = (jnp.dot(t, wb_ref[...], preferred_element_type=jnp.float32)
                  + bb_ref[...])


def _tc_mlp(eps, x, a0, a1, Wa, ba, Wb, bb):
    row_spec = pl.BlockSpec((ROW_BLK, D), lambda i: (i, 0))
    full_spec = pl.BlockSpec((D, D), lambda i: (0, 0))
    bias_spec = pl.BlockSpec((1, D), lambda i: (0, 0))
    return pl.pallas_call(
        _mlp_body,
        grid=(GRID,),
        in_specs=[
            pl.BlockSpec(memory_space=pltpu.SMEM),
            row_spec, row_spec, row_spec,
            full_spec, bias_spec, full_spec, bias_spec,
        ],
        out_specs=row_spec,
        out_shape=jax.ShapeDtypeStruct((N_NODES, D), jnp.float32),
    )(jnp.reshape(eps, (1, 1)), x, a0, a1,
      Wa, jnp.reshape(ba, (1, D)), Wb, jnp.reshape(bb, (1, D)))


def _final_body(x0_ref, x1_ref, x2_ref, x3_ref, w0_ref, w1_ref, w2_ref, w3_ref,
                bf_ref, o_ref):
    acc = jnp.dot(x0_ref[...], w0_ref[...], preferred_element_type=jnp.float32)
    acc += jnp.dot(x1_ref[...], w1_ref[...], preferred_element_type=jnp.float32)
    acc += jnp.dot(x2_ref[...], w2_ref[...], preferred_element_type=jnp.float32)
    acc += jnp.dot(x3_ref[...], w3_ref[...], preferred_element_type=jnp.float32)
    o_ref[...] = acc + bf_ref[...]


def _tc_final(x0, x1, x2, x3, Wf, bf):
    row_spec = pl.BlockSpec((ROW_BLK, D), lambda i: (i, 0))
    full_spec = pl.BlockSpec((D, D), lambda i: (0, 0))
    bias_spec = pl.BlockSpec((1, D), lambda i: (0, 0))
    return pl.pallas_call(
        _final_body,
        grid=(GRID,),
        in_specs=[row_spec, row_spec, row_spec, row_spec,
                  full_spec, full_spec, full_spec, full_spec, bias_spec],
        out_specs=row_spec,
        out_shape=jax.ShapeDtypeStruct((N_NODES, D), jnp.float32),
    )(x0, x1, x2, x3,
      Wf[0:D], Wf[D:2 * D], Wf[2 * D:3 * D], Wf[3 * D:4 * D],
      jnp.reshape(bf, (1, D)))


def kernel(x, edge_index,
           eps1, W1a, b1a, W1b, b1b,
           eps2, W2a, b2a, W2b, b2b,
           eps3, W3a, b3a, W3b, b3b,
           Wf, bf):
    src = edge_index[0].astype(jnp.int32).reshape(NW, NCHUNK, CHUNK)
    dst = edge_index[1].astype(jnp.int32).reshape(NW, NCHUNK, CHUNK)
    zeros_rows = jnp.zeros((ROWS_PER_SUB, D), jnp.float32)

    xs = [x]
    params = [(eps1, W1a, b1a, W1b, b1b),
              (eps2, W2a, b2a, W2b, b2b),
              (eps3, W3a, b3a, W3b, b3b)]
    for (eps, Wa, ba, Wb, bb) in params:
        partials = _sc_aggregate(xs[-1], src, dst, zeros_rows)
        xs.append(_tc_mlp(eps, xs[-1], partials[0], partials[1],
                          Wa, ba, Wb, bb))
    return _tc_final(xs[0], xs[1], xs[2], xs[3], Wf, bf)


# speedup vs baseline: 6.8722x; 6.8722x over previous
import functools

import jax
import jax.numpy as jnp
from jax import lax
from jax.experimental import pallas as pl
from jax.experimental.pallas import tpu as pltpu
from jax.experimental.pallas import tpu_sc as plsc

N_NODES = 10000
N_EDGES = 320000
D = 128

NC = 2
NS = 16
NW = NC * NS
E_PER_W = N_EDGES // NW
CHUNK = 80
NCHUNK = E_PER_W // CHUNK
N_PAD = 10240
ROWS_PER_SUB = N_PAD // NS


def _sc_aggregate(x, src, dst, zeros_rows):
    mesh = plsc.VectorSubcoreMesh(core_axis_name="c", subcore_axis_name="s")

    @functools.partial(
        pl.kernel,
        out_type=jax.ShapeDtypeStruct((NC, N_PAD, D), jnp.float32),
        mesh=mesh,
        scratch_types=[
            pltpu.VMEM((NCHUNK, CHUNK), jnp.int32),
            pltpu.VMEM((NCHUNK, CHUNK), jnp.int32),
            pltpu.VMEM((CHUNK, D), jnp.float32),
            pltpu.VMEM_SHARED((N_PAD, D), jnp.float32),
            pltpu.SemaphoreType.DMA,
        ],
    )
    def agg_kernel(x_hbm, src_hbm, dst_hbm, zeros_hbm, out_hbm,
                   src_v, dst_v, rows_v, acc_sh, sem):
        c = lax.axis_index("c")
        s = lax.axis_index("s")
        wid = c * NS + s

        pltpu.sync_copy(zeros_hbm, acc_sh.at[pl.ds(s * ROWS_PER_SUB, ROWS_PER_SUB)])
        pltpu.sync_copy(src_hbm.at[wid], src_v)
        pltpu.sync_copy(dst_hbm.at[wid], dst_v)
        plsc.subcore_barrier()

        def body(j, carry):
            pltpu.async_copy(x_hbm.at[src_v.at[j]], rows_v, sem).wait()
            pltpu.sync_copy(rows_v, acc_sh.at[dst_v.at[j]], add=True)
            return carry

        lax.fori_loop(0, NCHUNK, body, 0)
        plsc.subcore_barrier()

        pltpu.sync_copy(
            acc_sh.at[pl.ds(s * ROWS_PER_SUB, ROWS_PER_SUB)],
            out_hbm.at[c, pl.ds(s * ROWS_PER_SUB, ROWS_PER_SUB)],
        )

    return agg_kernel(x, src, dst, zeros_rows)


ROW_BLK = 2000
GRID = N_NODES // ROW_BLK


def _mlp_body(eps_ref, x_ref, a0_ref, a1_ref, wa_ref, ba_ref, wb_ref, bb_ref,
              o_ref):
    h = x_ref[...] * (1.0 + eps_ref[0, 0]) + a0_ref[...] + a1_ref[...]
    t = jnp.maximum(
        jnp.dot(h, wa_ref[...], preferred_element_type=jnp.float32)
        + ba_ref[...], 0.0)
    o_ref[...] = (jnp.dot(t, wb_ref[...], preferred_element_type=jnp.float32)
                  + bb_ref[...])


def _tc_mlp(eps, x, a0, a1, Wa, ba, Wb, bb):
    row_spec = pl.BlockSpec((ROW_BLK, D), lambda i: (i, 0))
    full_spec = pl.BlockSpec((D, D), lambda i: (0, 0))
    bias_spec = pl.BlockSpec((1, D), lambda i: (0, 0))
    return pl.pallas_call(
        _mlp_body,
        grid=(GRID,),
        in_specs=[
            pl.BlockSpec(memory_space=pltpu.SMEM),
            row_spec, row_spec, row_spec,
            full_spec, bias_spec, full_spec, bias_spec,
        ],
        out_specs=row_spec,
        out_shape=jax.ShapeDtypeStruct((N_NODES, D), jnp.float32),
    )(jnp.reshape(eps, (1, 1)), x, a0, a1,
      Wa, jnp.reshape(ba, (1, D)), Wb, jnp.reshape(bb, (1, D)))


def _final_body(x0_ref, x1_ref, x2_ref, x3_ref, w0_ref, w1_ref, w2_ref, w3_ref,
                bf_ref, o_ref):
    acc = jnp.dot(x0_ref[...], w0_ref[...], preferred_element_type=jnp.float32)
    acc += jnp.dot(x1_ref[...], w1_ref[...], preferred_element_type=jnp.float32)
    acc += jnp.dot(x2_ref[...], w2_ref[...], preferred_element_type=jnp.float32)
    acc += jnp.dot(x3_ref[...], w3_ref[...], preferred_element_type=jnp.float32)
    o_ref[...] = acc + bf_ref[...]


def _tc_final(x0, x1, x2, x3, Wf, bf):
    row_spec = pl.BlockSpec((ROW_BLK, D), lambda i: (i, 0))
    full_spec = pl.BlockSpec((D, D), lambda i: (0, 0))
    bias_spec = pl.BlockSpec((1, D), lambda i: (0, 0))
    return pl.pallas_call(
        _final_body,
        grid=(GRID,),
        in_specs=[row_spec, row_spec, row_spec, row_spec,
                  full_spec, full_spec, full_spec, full_spec, bias_spec],
        out_specs=row_spec,
        out_shape=jax.ShapeDtypeStruct((N_NODES, D), jnp.float32),
    )(x0, x1, x2, x3,
      Wf[0:D], Wf[D:2 * D], Wf[2 * D:3 * D], Wf[3 * D:4 * D],
      jnp.reshape(bf, (1, D)))


def kernel(x, edge_index,
           eps1, W1a, b1a, W1b, b1b,
           eps2, W2a, b2a, W2b, b2b,
           eps3, W3a, b3a, W3b, b3b,
           Wf, bf):
    src = edge_index[0].astype(jnp.int32).reshape(NW, NCHUNK, CHUNK)
    dst = edge_index[1].astype(jnp.int32).reshape(NW, NCHUNK, CHUNK)
    zeros_rows = jnp.zeros((ROWS_PER_SUB, D), jnp.float32)

    xs = [x]
    params = [(eps1, W1a, b1a, W1b, b1b),
              (eps2, W2a, b2a, W2b, b2b),
              (eps3, W3a, b3a, W3b, b3b)]
    for (eps, Wa, ba, Wb, bb) in params:
        partials = _sc_aggregate(xs[-1], src, dst, zeros_rows)
        xs.append(_tc_mlp(eps, xs[-1],
                          partials[0, :N_NODES], partials[1, :N_NODES],
                          Wa, ba, Wb, bb))
    return _tc_final(xs[0], xs[1], xs[2], xs[3], Wf, bf)


# speedup vs baseline: 10.2155x; 1.4865x over previous
import functools

import jax
import jax.numpy as jnp
from jax import lax
from jax.experimental import pallas as pl
from jax.experimental.pallas import tpu as pltpu
from jax.experimental.pallas import tpu_sc as plsc

N_NODES = 10000
N_EDGES = 320000
D = 128

NC = 2
NS = 16
NW = NC * NS
E_PER_W = N_EDGES // NW
CHUNK = 80
NCHUNK = E_PER_W // CHUNK
IBLK = 25
NBLK = NCHUNK // IBLK
N_PAD = 10240
ROWS_PER_SUB = N_PAD // NS


def _sc_aggregate(x, src, dst, zeros_rows):
    mesh = plsc.VectorSubcoreMesh(core_axis_name="c", subcore_axis_name="s")

    @functools.partial(
        pl.kernel,
        out_type=jax.ShapeDtypeStruct((NC, N_PAD, D), jnp.float32),
        mesh=mesh,
        scratch_types=[
            pltpu.VMEM((IBLK, CHUNK), jnp.int32),
            pltpu.VMEM((IBLK, CHUNK), jnp.int32),
            pltpu.VMEM((CHUNK, D), jnp.float32),
            pltpu.VMEM((CHUNK, D), jnp.float32),
            pltpu.VMEM_SHARED((N_PAD, D), jnp.float32),
            pltpu.SemaphoreType.DMA,
            pltpu.SemaphoreType.DMA,
        ],
    )
    def agg_kernel(x_hbm, src_hbm, dst_hbm, zeros_hbm, out_hbm,
                   src_v, dst_v, rows0, rows1, acc_sh, sem0, sem1):
        c = lax.axis_index("c")
        s = lax.axis_index("s")
        wid = c * NS + s

        pltpu.sync_copy(zeros_hbm, acc_sh.at[pl.ds(s * ROWS_PER_SUB, ROWS_PER_SUB)])
        plsc.subcore_barrier()

        def gather(j, buf, sem):
            pltpu.async_copy(x_hbm.at[src_v.at[j]], buf, sem)

        def gwait(buf, sem):
            pltpu.make_async_copy(x_hbm.at[src_v.at[0]], buf, sem).wait()

        def scatter(j, buf):
            pltpu.sync_copy(buf, acc_sh.at[dst_v.at[j]], add=True)

        NPAIR = (IBLK - 1) // 2

        def block(k, carry):
            pltpu.sync_copy(src_hbm.at[wid, k], src_v)
            pltpu.sync_copy(dst_hbm.at[wid, k], dst_v)
            gather(0, rows0, sem0)

            def body(t, c2):
                j0 = 2 * t
                gather(j0 + 1, rows1, sem1)
                gwait(rows0, sem0)
                scatter(j0, rows0)
                gather(j0 + 2, rows0, sem0)
                gwait(rows1, sem1)
                scatter(j0 + 1, rows1)
                return c2

            lax.fori_loop(0, NPAIR, body, 0)
            gwait(rows0, sem0)
            scatter(IBLK - 1, rows0)
            return carry

        lax.fori_loop(0, NBLK, block, 0)
        plsc.subcore_barrier()

        pltpu.sync_copy(
            acc_sh.at[pl.ds(s * ROWS_PER_SUB, ROWS_PER_SUB)],
            out_hbm.at[c, pl.ds(s * ROWS_PER_SUB, ROWS_PER_SUB)],
        )

    return agg_kernel(x, src, dst, zeros_rows)


ROW_BLK = 2000
GRID = N_NODES // ROW_BLK


def _mlp_body(eps_ref, x_ref, a0_ref, a1_ref, wa_ref, ba_ref, wb_ref, bb_ref,
              o_ref):
    h = x_ref[...] * (1.0 + eps_ref[0, 0]) + a0_ref[...] + a1_ref[...]
    t = jnp.maximum(
        jnp.dot(h, wa_ref[...], preferred_element_type=jnp.float32)
        + ba_ref[...], 0.0)
    o_ref[...] = (jnp.dot(t, wb_ref[...], preferred_element_type=jnp.float32)
                  + bb_ref[...])


def _tc_mlp(eps, x, a0, a1, Wa, ba, Wb, bb):
    row_spec = pl.BlockSpec((ROW_BLK, D), lambda i: (i, 0))
    full_spec = pl.BlockSpec((D, D), lambda i: (0, 0))
    bias_spec = pl.BlockSpec((1, D), lambda i: (0, 0))
    return pl.pallas_call(
        _mlp_body,
        grid=(GRID,),
        in_specs=[
            pl.BlockSpec(memory_space=pltpu.SMEM),
            row_spec, row_spec, row_spec,
            full_spec, bias_spec, full_spec, bias_spec,
        ],
        out_specs=row_spec,
        out_shape=jax.ShapeDtypeStruct((N_NODES, D), jnp.float32),
    )(jnp.reshape(eps, (1, 1)), x, a0, a1,
      Wa, jnp.reshape(ba, (1, D)), Wb, jnp.reshape(bb, (1, D)))


def _final_body(x0_ref, x1_ref, x2_ref, x3_ref, w0_ref, w1_ref, w2_ref, w3_ref,
                bf_ref, o_ref):
    acc = jnp.dot(x0_ref[...], w0_ref[...], preferred_element_type=jnp.float32)
    acc += jnp.dot(x1_ref[...], w1_ref[...], preferred_element_type=jnp.float32)
    acc += jnp.dot(x2_ref[...], w2_ref[...], preferred_element_type=jnp.float32)
    acc += jnp.dot(x3_ref[...], w3_ref[...], preferred_element_type=jnp.float32)
    o_ref[...] = acc + bf_ref[...]


def _tc_final(x0, x1, x2, x3, Wf, bf):
    row_spec = pl.BlockSpec((ROW_BLK, D), lambda i: (i, 0))
    full_spec = pl.BlockSpec((D, D), lambda i: (0, 0))
    bias_spec = pl.BlockSpec((1, D), lambda i: (0, 0))
    return pl.pallas_call(
        _final_body,
        grid=(GRID,),
        in_specs=[row_spec, row_spec, row_spec, row_spec,
                  full_spec, full_spec, full_spec, full_spec, bias_spec],
        out_specs=row_spec,
        out_shape=jax.ShapeDtypeStruct((N_NODES, D), jnp.float32),
    )(x0, x1, x2, x3,
      Wf[0:D], Wf[D:2 * D], Wf[2 * D:3 * D], Wf[3 * D:4 * D],
      jnp.reshape(bf, (1, D)))


def kernel(x, edge_index,
           eps1, W1a, b1a, W1b, b1b,
           eps2, W2a, b2a, W2b, b2b,
           eps3, W3a, b3a, W3b, b3b,
           Wf, bf):
    src = edge_index[0].astype(jnp.int32).reshape(NW, NBLK, IBLK, CHUNK)
    dst = edge_index[1].astype(jnp.int32).reshape(NW, NBLK, IBLK, CHUNK)
    zeros_rows = jnp.zeros((ROWS_PER_SUB, D), jnp.float32)

    xs = [x]
    params = [(eps1, W1a, b1a, W1b, b1b),
              (eps2, W2a, b2a, W2b, b2b),
              (eps3, W3a, b3a, W3b, b3b)]
    for (eps, Wa, ba, Wb, bb) in params:
        partials = _sc_aggregate(xs[-1], src, dst, zeros_rows)
        xs.append(_tc_mlp(eps, xs[-1],
                          partials[0, :N_NODES], partials[1, :N_NODES],
                          Wa, ba, Wb, bb))
    return _tc_final(xs[0], xs[1], xs[2], xs[3], Wf, bf)
